# baseline (speedup 1.0000x reference)
import functools

import jax
import jax.numpy as jnp
from jax import lax
from jax.experimental import pallas as pl
from jax.experimental.pallas import tpu as pltpu
from jax.experimental.pallas import tpu_sc as plsc

N, K, C = 10000, 27, 256
KP = 28
RB = 4
NC, NS = 2, 16
NW = NC * NS
NCHUNK = (N + RB - 1) // RB
CPW = -(-NCHUNK // NW)
CPW += CPW % 2
NPAD = NW * CPW * RB
CW = RB * KP
ZR = N
LANES = 16


def _dwconv_body(data_hbm, nflat_hbm, w_hbm, out_hbm,
                 idx_v, idxb_v, g0a, g0b, g1a, g1b, w_v, out_v, sem0, sem1):
    wid = lax.axis_index("c") * NS + lax.axis_index("s")
    base0 = wid * CPW
    nvalid = jnp.minimum(CPW, jnp.maximum(NCHUNK - base0, 0))

    pltpu.sync_copy(w_hbm, w_v)
    pltpu.sync_copy(nflat_hbm.at[pl.ds(wid * CPW, CPW)], idx_v)

    def _clean(i, carry):
        for u in range(CW // LANES):
            v = idx_v[i, pl.ds(u * LANES, LANES)]
            v2 = jnp.where(v < 0, ZR, v) * 2
            idx_v[i, pl.ds(u * LANES, LANES)] = v2
            idxb_v[i, pl.ds(u * LANES, LANES)] = v2 + 1
        return carry
    lax.fori_loop(0, CPW, _clean, 0)

    H = C // 2

    def _gather_descs(j, ga, gb, sem):
        return (
            pltpu.make_async_copy(data_hbm.at[idx_v.at[j]], ga, sem),
            pltpu.make_async_copy(data_hbm.at[idxb_v.at[j]], gb, sem),
        )

    def _gather_start(j, ga, gb, sem):
        da, db = _gather_descs(j, ga, gb, sem)
        da.start()
        db.start()

    def _gather_wait(j, ga, gb, sem):
        da, db = _gather_descs(j, ga, gb, sem)
        da.wait()
        db.wait()

    def _compute(ga, gb):
        def half(gbuf, coff):
            def c_body(ci, carry):
                cs = ci * LANES
                wr = [w_v[k, pl.ds(coff + cs, LANES)] for k in range(K)]
                for r in range(RB):
                    acc = gbuf[r * KP, pl.ds(cs, LANES)] * wr[0]
                    for k in range(1, K):
                        acc = acc + gbuf[r * KP + k, pl.ds(cs, LANES)] * wr[k]
                    out_v[r, pl.ds(coff + cs, LANES)] = acc
                return carry
            lax.fori_loop(0, H // LANES, c_body, 0)
        half(ga, 0)
        half(gb, H)

    @pl.when(nvalid > 0)
    def _prime0():
        _gather_start(0, g0a, g0b, sem0)

    @pl.when(nvalid > 1)
    def _prime1():
        _gather_start(1, g1a, g1b, sem1)

    def _pair(j2, carry):
        for b, (ga, gb, sem) in enumerate(((g0a, g0b, sem0),
                                           (g1a, g1b, sem1))):
            j = j2 * 2 + b

            @pl.when(j < nvalid)
            def _do():
                _gather_wait(j, ga, gb, sem)
                _compute(ga, gb)
                pltpu.sync_copy(out_v, out_hbm.at[pl.ds((base0 + j) * RB, RB)])

                @pl.when(j + 2 < nvalid)
                def _next():
                    _gather_start(j + 2, ga, gb, sem)
        return carry
    lax.fori_loop(0, CPW // 2, _pair, 0)


@functools.cache
def _dwconv():
    return functools.partial(
        pl.kernel,
        out_type=jax.ShapeDtypeStruct((N, C), jnp.float32),
        mesh=plsc.VectorSubcoreMesh(core_axis_name="c", subcore_axis_name="s",
                                    num_cores=NC, num_subcores=NS),
        compiler_params=pltpu.CompilerParams(use_tc_tiling_on_sc=False),
        scratch_types=[
            pltpu.VMEM((CPW, CW), jnp.int32),
            pltpu.VMEM((CPW, CW), jnp.int32),
            pltpu.VMEM((RB * KP, C // 2), jnp.float32),
            pltpu.VMEM((RB * KP, C // 2), jnp.float32),
            pltpu.VMEM((RB * KP, C // 2), jnp.float32),
            pltpu.VMEM((RB * KP, C // 2), jnp.float32),
            pltpu.VMEM((K, C), jnp.float32),
            pltpu.VMEM((RB, C), jnp.float32),
            pltpu.SemaphoreType.DMA,
            pltpu.SemaphoreType.DMA,
        ],
    )(_dwconv_body)


def kernel(data, neigh, weights):
    data_p = jnp.concatenate(
        [data, jnp.zeros((8, C), jnp.float32)], axis=0).reshape(-1, C // 2)
    n28 = jnp.concatenate(
        [neigh, jnp.full((N, KP - K), -1, jnp.int32)], axis=1)
    nflat = jnp.concatenate(
        [n28, jnp.full((NPAD - N, KP), -1, jnp.int32)], axis=0).reshape(
            NW * CPW, CW)
    return _dwconv()(data_p, nflat, weights.reshape(K, C))


# speedup vs baseline: 2.0080x; 2.0080x over previous
import functools

import jax
import jax.numpy as jnp
from jax import lax
from jax.experimental import pallas as pl
from jax.experimental.pallas import tpu as pltpu
from jax.experimental.pallas import tpu_sc as plsc

N, K, C = 10000, 27, 256
NP = N + 8
NC, NS = 2, 16
NW = NC * NS
CPT = C // NW
RBK = 512
NBLK = -(-N // RBK)
ZR = N
LANES = 16


def _dwconv_body(data_hbm, neigh_hbm, w_hbm, out_hbm,
                 tab_v, i0, i1, ob0, ob1, w_v,
                 si0, si1, so0, so1):
    wid = lax.axis_index("c") * NS + lax.axis_index("s")
    cbase = wid * CPT

    pltpu.sync_copy(data_hbm.at[:, pl.ds(cbase, CPT)], tab_v.at[pl.ds(0, N)])
    pltpu.sync_copy(w_hbm.at[pl.ds(cbase, CPT)], w_v)

    iota16 = lax.iota(jnp.int32, LANES)
    zrows = (iota16 >> 3) + N
    zcols = iota16 & 7
    for j in range(4):
        plsc.store_scatter(tab_v, [zrows + 2 * j, zcols],
                           jnp.zeros((LANES,), jnp.float32))

    def _sb(b):
        return jnp.minimum(b * RBK, N - RBK)

    def _idx_copy(b, ibuf, sem):
        return pltpu.make_async_copy(
            neigh_hbm.at[pl.ds(_sb(b), RBK)], ibuf, sem)

    def _out_copy(b, obuf, sem):
        return pltpu.make_async_copy(
            obuf, out_hbm.at[pl.ds(_sb(b), RBK), pl.ds(cbase, CPT)], sem)

    def _compute(ibuf, obuf):
        for cp in range(CPT // 2):
            c0, c1 = 2 * cp, 2 * cp + 1
            w0 = [w_v[c0, k] for k in range(K)]
            w1 = [w_v[c1, k] for k in range(K)]
            cv0 = jnp.full((LANES,), c0, jnp.int32)
            cv1 = jnp.full((LANES,), c1, jnp.int32)

            def iblk(ii, carry):
                rows = iota16 + ii * LANES

                def ld(k):
                    kv = jnp.full((LANES,), k, jnp.int32)
                    ivr = plsc.load_gather(ibuf, [rows, kv])
                    return jnp.where(ivr < 0, ZR, ivr)

                iv = ld(0)
                acc0 = plsc.load_gather(tab_v, [iv, cv0]) * w0[0]
                acc1 = plsc.load_gather(tab_v, [iv, cv1]) * w1[0]
                for k in range(1, K):
                    iv = ld(k)
                    acc0 = acc0 + plsc.load_gather(tab_v, [iv, cv0]) * w0[k]
                    acc1 = acc1 + plsc.load_gather(tab_v, [iv, cv1]) * w1[k]
                plsc.store_scatter(obuf, [rows, cv0], acc0)
                plsc.store_scatter(obuf, [rows, cv1], acc1)
                return carry
            lax.fori_loop(0, RBK // LANES, iblk, 0)

    _idx_copy(0, i0, si0).start()
    _idx_copy(0, i0, si0).wait()
    _idx_copy(1, i1, si1).start()

    def _pair(b2, carry):
        for parity, (ibuf, isem, obuf, osem) in enumerate(
                ((i0, si0, ob0, so0), (i1, si1, ob1, so1))):
            b = b2 * 2 + parity

            @pl.when(b > 0)
            def _wait_idx():
                _idx_copy(b, ibuf, isem).wait()

            @pl.when(b >= 2)
            def _wait_out():
                _out_copy(b - 2, obuf, osem).wait()

            _compute(ibuf, obuf)

            @pl.when(b + 2 < NBLK)
            def _next_idx():
                _idx_copy(b + 2, ibuf, isem).start()

            _out_copy(b, obuf, osem).start()
        return carry
    lax.fori_loop(0, NBLK // 2, _pair, 0)

    _out_copy(NBLK - 2, ob0, so0).wait()
    _out_copy(NBLK - 1, ob1, so1).wait()


@functools.cache
def _dwconv():
    return functools.partial(
        pl.kernel,
        out_type=jax.ShapeDtypeStruct((N, C), jnp.float32),
        mesh=plsc.VectorSubcoreMesh(core_axis_name="c", subcore_axis_name="s",
                                    num_cores=NC, num_subcores=NS),
        compiler_params=pltpu.CompilerParams(use_tc_tiling_on_sc=False,
                                             needs_layout_passes=False),
        scratch_types=[
            pltpu.VMEM((NP, CPT), jnp.float32),
            pltpu.VMEM((RBK, K), jnp.int32),
            pltpu.VMEM((RBK, K), jnp.int32),
            pltpu.VMEM((RBK, CPT), jnp.float32),
            pltpu.VMEM((RBK, CPT), jnp.float32),
            pltpu.VMEM((CPT, K, LANES), jnp.float32),
            pltpu.SemaphoreType.DMA,
            pltpu.SemaphoreType.DMA,
            pltpu.SemaphoreType.DMA,
            pltpu.SemaphoreType.DMA,
        ],
    )(_dwconv_body)


def kernel(data, neigh, weights):
    w_splat = jnp.broadcast_to(
        weights.reshape(K, C).T.reshape(C, K, 1), (C, K, LANES))
    return _dwconv()(data, neigh, w_splat)


# speedup vs baseline: 3.1579x; 1.5726x over previous
import functools

import jax
import jax.numpy as jnp
from jax import lax
from jax.experimental import pallas as pl
from jax.experimental.pallas import tpu as pltpu
from jax.experimental.pallas import tpu_sc as plsc

N, K, C = 10000, 27, 256
NP = N + 8
NC, NS = 2, 16
NW = NC * NS
CPT = C // NW
RBK = 512
NBLK = -(-N // RBK)
ZR = N
LANES = 16


def _dwconv_body(data_hbm, neigh_hbm, w_hbm, out_hbm,
                 tab_v, i0, i1, ob0, w_v, spo,
                 si0, si1, so):
    cid = lax.axis_index("c")
    sid = lax.axis_index("s")
    scbase = cid * (NS * CPT)
    cbase = scbase + sid * CPT

    pltpu.sync_copy(data_hbm.at[:, pl.ds(cbase, CPT)], tab_v.at[pl.ds(0, N)])
    pltpu.sync_copy(w_hbm.at[pl.ds(cbase, CPT)], w_v)

    iota16 = lax.iota(jnp.int32, LANES)
    zrows = (iota16 >> 3) + N
    zcols = iota16 & 7
    for j in range(4):
        plsc.store_scatter(tab_v, [zrows + 2 * j, zcols],
                           jnp.zeros((LANES,), jnp.float32))

    def _sb(b):
        return jnp.minimum(b * RBK, N - RBK)

    def _idx_copy(b, ibuf, sem):
        return pltpu.make_async_copy(neigh_hbm.at[b], ibuf, sem)

    def _clean(ibuf):
        def body(i, carry):
            for u in range(RBK // LANES):
                v = ibuf[i, pl.ds(u * LANES, LANES)]
                ibuf[i, pl.ds(u * LANES, LANES)] = jnp.where(v < 0, ZR, v)
            return carry
        lax.fori_loop(0, K, body, 0)

    def _out_dma(b, spob, sem):
        return pltpu.make_async_copy(
            spob, out_hbm.at[pl.ds(_sb(b), RBK), pl.ds(scbase, NS * CPT)],
            sem)

    def _compute(ibuf, obuf):
        for cp in range(CPT // 2):
            c0, c1 = 2 * cp, 2 * cp + 1
            w0 = [w_v[c0, k] for k in range(K)]
            w1 = [w_v[c1, k] for k in range(K)]
            cv0 = jnp.full((LANES,), c0, jnp.int32)
            cv1 = jnp.full((LANES,), c1, jnp.int32)

            def iblk(ii, carry):
                base = ii * LANES
                rows = iota16 + base
                iv = ibuf[0, pl.ds(base, LANES)]
                acc0 = plsc.load_gather(tab_v, [iv, cv0]) * w0[0]
                acc1 = plsc.load_gather(tab_v, [iv, cv1]) * w1[0]
                for k in range(1, K):
                    iv = ibuf[k, pl.ds(base, LANES)]
                    acc0 = acc0 + plsc.load_gather(tab_v, [iv, cv0]) * w0[k]
                    acc1 = acc1 + plsc.load_gather(tab_v, [iv, cv1]) * w1[k]
                plsc.store_scatter(obuf, [rows, cv0], acc0)
                plsc.store_scatter(obuf, [rows, cv1], acc1)
                return carry
            lax.fori_loop(0, RBK // LANES, iblk, 0)

    _idx_copy(0, i0, si0).start()
    _idx_copy(0, i0, si0).wait()
    _idx_copy(1, i1, si1).start()

    def _pair(b2, carry):
        for parity, (ibuf, isem, obuf) in enumerate(
                ((i0, si0, ob0), (i1, si1, ob0))):
            b = b2 * 2 + parity

            @pl.when(b > 0)
            def _wait_idx():
                _idx_copy(b, ibuf, isem).wait()

            _clean(ibuf)
            _compute(ibuf, obuf)

            @pl.when(b + 2 < NBLK)
            def _next_idx():
                _idx_copy(b + 2, ibuf, isem).start()

            @pl.when((b > 0) & (sid == 0))
            def _wait_out():
                _out_dma(b - 1, spo, so).wait()
            plsc.subcore_barrier()
            pltpu.sync_copy(obuf, spo.at[:, pl.ds(sid * CPT, CPT)])
            plsc.subcore_barrier()

            @pl.when(sid == 0)
            def _start_out():
                _out_dma(b, spo, so).start()
        return carry
    lax.fori_loop(0, NBLK // 2, _pair, 0)

    @pl.when(sid == 0)
    def _drain():
        _out_dma(NBLK - 1, spo, so).wait()


@functools.cache
def _dwconv():
    return functools.partial(
        pl.kernel,
        out_type=jax.ShapeDtypeStruct((N, C), jnp.float32),
        mesh=plsc.VectorSubcoreMesh(core_axis_name="c", subcore_axis_name="s",
                                    num_cores=NC, num_subcores=NS),
        compiler_params=pltpu.CompilerParams(use_tc_tiling_on_sc=False,
                                             needs_layout_passes=False),
        scratch_types=[
            pltpu.VMEM((NP, CPT), jnp.float32),
            pltpu.VMEM((K, RBK), jnp.int32),
            pltpu.VMEM((K, RBK), jnp.int32),
            pltpu.VMEM((RBK, CPT), jnp.float32),
            pltpu.VMEM((CPT, K, LANES), jnp.float32),
            pltpu.VMEM_SHARED((RBK, NS * CPT), jnp.float32),
            pltpu.SemaphoreType.DMA,
            pltpu.SemaphoreType.DMA,
            pltpu.SemaphoreType.DMA,
        ],
    )(_dwconv_body)


def kernel(data, neigh, weights):
    starts = jnp.minimum(jnp.arange(NBLK) * RBK, N - RBK)
    rows = (starts[:, None] + jnp.arange(RBK)[None, :]).reshape(-1)
    neighb = neigh[rows].reshape(NBLK, RBK, K).transpose(0, 2, 1)
    w_splat = jnp.broadcast_to(
        weights.reshape(K, C).T.reshape(C, K, 1), (C, K, LANES))
    return _dwconv()(data, neighb, w_splat)
